# Initial kernel scaffold; baseline (speedup 1.0000x reference)
import jax
import jax.numpy as jnp
from jax.experimental import pallas as pl


def kernel(x, pos, edge_index, l0_lw1, l0_lb1, l0_lw2, l0_lb2, l0_gw1, l0_gb1, l0_gw2, l0_gb2, l1_lw1, l1_lb1, l1_lw2, l1_lb2, l1_gw1, l1_gb1, l1_gw2, l1_gb2):
    raise NotImplementedError("write your pallas kernel here")


# speedup vs baseline: 1.0457x; 1.0457x over previous
import functools

import jax
import jax.numpy as jnp
from jax import lax
from jax.experimental import pallas as pl
from jax.experimental.pallas import tpu as pltpu

N = 10000
HID = 128
POS_DIM = 100

_NODE_BLK = 1000
_EDGE_BLK = 2000


def _ab_kernel(h_ref, pos_ref, wx_ref, wp_ref, b1_ref, a_ref, b_ref):
    pos_proj = jax.lax.dot_general(
        pos_ref[...], wp_ref[...], (((1,), (1,)), ((), ())),
        preferred_element_type=jnp.float32)
    b_ref[...] = pos_proj
    a_ref[...] = jax.lax.dot_general(
        h_ref[...], wx_ref[...], (((1,), (1,)), ((), ())),
        preferred_element_type=jnp.float32) + pos_proj + b1_ref[...]


def _node_precompute(h, pos, lw1, lb1):
    ind = h.shape[1]
    wx = lw1[:, :ind]
    wp = lw1[:, ind:]
    grid = N // _NODE_BLK
    return pl.pallas_call(
        _ab_kernel,
        grid=(grid,),
        in_specs=[
            pl.BlockSpec((_NODE_BLK, ind), lambda i: (i, 0)),
            pl.BlockSpec((_NODE_BLK, POS_DIM), lambda i: (i, 0)),
            pl.BlockSpec((HID, ind), lambda i: (0, 0)),
            pl.BlockSpec((HID, POS_DIM), lambda i: (0, 0)),
            pl.BlockSpec((1, HID), lambda i: (0, 0)),
        ],
        out_specs=[
            pl.BlockSpec((_NODE_BLK, HID), lambda i: (i, 0)),
            pl.BlockSpec((_NODE_BLK, HID), lambda i: (i, 0)),
        ],
        out_shape=[
            jax.ShapeDtypeStruct((N, HID), jnp.float32),
            jax.ShapeDtypeStruct((N, HID), jnp.float32),
        ],
    )(h, pos, wx, wp, lb1.reshape(1, HID))


def _edge_mm_kernel(asrc_ref, bdst_ref, w2_ref, m_ref):
    z = jnp.maximum(asrc_ref[...] - bdst_ref[...], 0.0)
    m_ref[...] = jax.lax.dot_general(
        z, w2_ref[...], (((1,), (1,)), ((), ())),
        preferred_element_type=jnp.float32)


def _edge_matmul(a_src, b_dst, lw2):
    e = a_src.shape[0]
    grid = e // _EDGE_BLK
    return pl.pallas_call(
        _edge_mm_kernel,
        grid=(grid,),
        in_specs=[
            pl.BlockSpec((_EDGE_BLK, HID), lambda i: (i, 0)),
            pl.BlockSpec((_EDGE_BLK, HID), lambda i: (i, 0)),
            pl.BlockSpec((HID, HID), lambda i: (0, 0)),
        ],
        out_specs=pl.BlockSpec((_EDGE_BLK, HID), lambda i: (i, 0)),
        out_shape=jax.ShapeDtypeStruct((e, HID), jnp.float32),
    )(a_src, b_dst, lw2)


def _global_mlp_kernel(aggr_ref, b2_ref, gw1_ref, gb1_ref, gw2_ref, gb2_ref, o_ref):
    a = aggr_ref[...] + b2_ref[...]
    t = jnp.maximum(
        jax.lax.dot_general(a, gw1_ref[...], (((1,), (1,)), ((), ())),
                            preferred_element_type=jnp.float32) + gb1_ref[...], 0.0)
    o_ref[...] = jax.lax.dot_general(
        t, gw2_ref[...], (((1,), (1,)), ((), ())),
        preferred_element_type=jnp.float32) + gb2_ref[...]


def _global_mlp(aggr, lb2, gw1, gb1, gw2, gb2):
    grid = N // _NODE_BLK
    return pl.pallas_call(
        _global_mlp_kernel,
        grid=(grid,),
        in_specs=[
            pl.BlockSpec((_NODE_BLK, HID), lambda i: (i, 0)),
            pl.BlockSpec((1, HID), lambda i: (0, 0)),
            pl.BlockSpec((HID, HID), lambda i: (0, 0)),
            pl.BlockSpec((1, HID), lambda i: (0, 0)),
            pl.BlockSpec((HID, HID), lambda i: (0, 0)),
            pl.BlockSpec((1, HID), lambda i: (0, 0)),
        ],
        out_specs=pl.BlockSpec((_NODE_BLK, HID), lambda i: (i, 0)),
        out_shape=jax.ShapeDtypeStruct((N, HID), jnp.float32),
    )(aggr, lb2.reshape(1, HID), gw1, gb1.reshape(1, HID), gw2,
      gb2.reshape(1, HID))


def _layer(h, pos, src, dst, lw1, lb1, lw2, lb2, gw1, gb1, gw2, gb2):
    a, b = _node_precompute(h, pos, lw1, lb1)
    a_src = a[src]
    b_dst = b[dst]
    m = _edge_matmul(a_src, b_dst, lw2)
    aggr = jax.ops.segment_max(m, dst, num_segments=N)
    return _global_mlp(aggr, lb2, gw1, gb1, gw2, gb2)


def kernel(x, pos, edge_index, l0_lw1, l0_lb1, l0_lw2, l0_lb2, l0_gw1,
           l0_gb1, l0_gw2, l0_gb2, l1_lw1, l1_lb1, l1_lw2, l1_lb2, l1_gw1,
           l1_gb1, l1_gw2, l1_gb2):
    n = x.shape[0]
    loop = jnp.arange(n, dtype=edge_index.dtype)
    src = jnp.concatenate([edge_index[0], loop])
    dst = jnp.concatenate([edge_index[1], loop])
    h = _layer(x, pos, src, dst, l0_lw1, l0_lb1, l0_lw2, l0_lb2,
               l0_gw1, l0_gb1, l0_gw2, l0_gb2)
    h = _layer(h, pos, src, dst, l1_lw1, l1_lb1, l1_lw2, l1_lb2,
               l1_gw1, l1_gb1, l1_gw2, l1_gb2)
    return h


# speedup vs baseline: 1.7999x; 1.7213x over previous
import functools

import jax
import jax.numpy as jnp
from jax import lax
from jax.experimental import pallas as pl
from jax.experimental.pallas import tpu as pltpu
from jax.experimental.pallas import tpu_sc as plsc

N = 10000
HID = 128
POS_DIM = 100

_NODE_BLK = 1000
_EDGE_BLK = 2048

_NC = 2
_NS = 16
_NW = _NC * _NS
_E2 = 337920
_CH = 120
_BPW = _E2 // _NW
_NCHUNK = _BPW // _CH


def _gather_z(a, b, src2d, dst2d):
    mesh = plsc.VectorSubcoreMesh(core_axis_name="c", subcore_axis_name="s")

    @functools.partial(
        pl.kernel,
        out_type=jax.ShapeDtypeStruct((_E2, HID), jnp.float32),
        mesh=mesh,
        scratch_types=[
            pltpu.VMEM((_NCHUNK, _CH), jnp.int32),
            pltpu.VMEM((_NCHUNK, _CH), jnp.int32),
            pltpu.VMEM((_CH, HID), jnp.float32),
            pltpu.VMEM((_CH, HID), jnp.float32),
            pltpu.SemaphoreType.DMA,
            pltpu.SemaphoreType.DMA,
        ],
    )
    def k(a_hbm, b_hbm, src_hbm, dst_hbm, z_hbm, sidx, didx, arows, brows,
          sa, sb):
        wid = lax.axis_index("s") * _NC + lax.axis_index("c")
        rowbase = wid * _NCHUNK
        ebase = wid * _BPW
        pltpu.sync_copy(src_hbm.at[pl.ds(rowbase, _NCHUNK)], sidx)
        pltpu.sync_copy(dst_hbm.at[pl.ds(rowbase, _NCHUNK)], didx)

        def chunk(i, carry):
            ca = pltpu.async_copy(a_hbm.at[sidx.at[i]], arows, sa)
            cb = pltpu.async_copy(b_hbm.at[didx.at[i]], brows, sb)
            ca.wait()
            cb.wait()

            def row(r, c2):
                for cc in range(HID // 16):
                    av = arows[r, pl.ds(cc * 16, 16)]
                    bv = brows[r, pl.ds(cc * 16, 16)]
                    arows[r, pl.ds(cc * 16, 16)] = jnp.maximum(av - bv, 0.0)
                return c2

            lax.fori_loop(0, _CH, row, 0)
            pltpu.sync_copy(arows, z_hbm.at[pl.ds(ebase + i * _CH, _CH)])
            return carry

        lax.fori_loop(0, _NCHUNK, chunk, 0)

    return k(a, b, src2d, dst2d)


def _ab_kernel(h_ref, pos_ref, wx_ref, wp_ref, b1_ref, a_ref, b_ref):
    pos_proj = jax.lax.dot_general(
        pos_ref[...], wp_ref[...], (((1,), (1,)), ((), ())),
        preferred_element_type=jnp.float32)
    b_ref[...] = pos_proj
    a_ref[...] = jax.lax.dot_general(
        h_ref[...], wx_ref[...], (((1,), (1,)), ((), ())),
        preferred_element_type=jnp.float32) + pos_proj + b1_ref[...]


def _node_precompute(h, pos, lw1, lb1):
    ind = h.shape[1]
    wx = lw1[:, :ind]
    wp = lw1[:, ind:]
    grid = N // _NODE_BLK
    return pl.pallas_call(
        _ab_kernel,
        grid=(grid,),
        in_specs=[
            pl.BlockSpec((_NODE_BLK, ind), lambda i: (i, 0)),
            pl.BlockSpec((_NODE_BLK, POS_DIM), lambda i: (i, 0)),
            pl.BlockSpec((HID, ind), lambda i: (0, 0)),
            pl.BlockSpec((HID, POS_DIM), lambda i: (0, 0)),
            pl.BlockSpec((1, HID), lambda i: (0, 0)),
        ],
        out_specs=[
            pl.BlockSpec((_NODE_BLK, HID), lambda i: (i, 0)),
            pl.BlockSpec((_NODE_BLK, HID), lambda i: (i, 0)),
        ],
        out_shape=[
            jax.ShapeDtypeStruct((N, HID), jnp.float32),
            jax.ShapeDtypeStruct((N, HID), jnp.float32),
        ],
    )(h, pos, wx, wp, lb1.reshape(1, HID))


def _edge_mm_kernel(z_ref, w2_ref, m_ref):
    m_ref[...] = jax.lax.dot_general(
        z_ref[...], w2_ref[...], (((1,), (1,)), ((), ())),
        preferred_element_type=jnp.float32)


def _edge_matmul(z, lw2):
    e = z.shape[0]
    grid = e // _EDGE_BLK
    return pl.pallas_call(
        _edge_mm_kernel,
        grid=(grid,),
        in_specs=[
            pl.BlockSpec((_EDGE_BLK, HID), lambda i: (i, 0)),
            pl.BlockSpec((HID, HID), lambda i: (0, 0)),
        ],
        out_specs=pl.BlockSpec((_EDGE_BLK, HID), lambda i: (i, 0)),
        out_shape=jax.ShapeDtypeStruct((e, HID), jnp.float32),
    )(z, lw2)


def _global_mlp_kernel(aggr_ref, b2_ref, gw1_ref, gb1_ref, gw2_ref, gb2_ref, o_ref):
    a = aggr_ref[...] + b2_ref[...]
    t = jnp.maximum(
        jax.lax.dot_general(a, gw1_ref[...], (((1,), (1,)), ((), ())),
                            preferred_element_type=jnp.float32) + gb1_ref[...], 0.0)
    o_ref[...] = jax.lax.dot_general(
        t, gw2_ref[...], (((1,), (1,)), ((), ())),
        preferred_element_type=jnp.float32) + gb2_ref[...]


def _global_mlp(aggr, lb2, gw1, gb1, gw2, gb2):
    grid = N // _NODE_BLK
    return pl.pallas_call(
        _global_mlp_kernel,
        grid=(grid,),
        in_specs=[
            pl.BlockSpec((_NODE_BLK, HID), lambda i: (i, 0)),
            pl.BlockSpec((1, HID), lambda i: (0, 0)),
            pl.BlockSpec((HID, HID), lambda i: (0, 0)),
            pl.BlockSpec((1, HID), lambda i: (0, 0)),
            pl.BlockSpec((HID, HID), lambda i: (0, 0)),
            pl.BlockSpec((1, HID), lambda i: (0, 0)),
        ],
        out_specs=pl.BlockSpec((_NODE_BLK, HID), lambda i: (i, 0)),
        out_shape=jax.ShapeDtypeStruct((N, HID), jnp.float32),
    )(aggr, lb2.reshape(1, HID), gw1, gb1.reshape(1, HID), gw2,
      gb2.reshape(1, HID))


def _layer(h, pos, src2d, dstg2d, dst_s, lw1, lb1, lw2, lb2, gw1, gb1, gw2,
           gb2):
    a, b = _node_precompute(h, pos, lw1, lb1)
    z = _gather_z(a, b, src2d, dstg2d)
    m = _edge_matmul(z, lw2)
    aggr = jax.ops.segment_max(m, dst_s, num_segments=N + 1)[:N]
    return _global_mlp(aggr, lb2, gw1, gb1, gw2, gb2)


def kernel(x, pos, edge_index, l0_lw1, l0_lb1, l0_lw2, l0_lb2, l0_gw1,
           l0_gb1, l0_gw2, l0_gb2, l1_lw1, l1_lb1, l1_lw2, l1_lb2, l1_gw1,
           l1_gb1, l1_gw2, l1_gb2):
    n = x.shape[0]
    e = edge_index.shape[1]
    npad = _E2 - e - n
    loop = jnp.arange(n, dtype=edge_index.dtype)
    src = jnp.concatenate(
        [edge_index[0], loop, jnp.zeros((npad,), edge_index.dtype)])
    dst = jnp.concatenate([edge_index[1], loop])
    dst_g = jnp.concatenate([dst, jnp.zeros((npad,), edge_index.dtype)])
    dst_s = jnp.concatenate([dst, jnp.full((npad,), n, edge_index.dtype)])
    src2d = src.reshape(_E2 // _CH, _CH)
    dstg2d = dst_g.reshape(_E2 // _CH, _CH)
    h = _layer(x, pos, src2d, dstg2d, dst_s, l0_lw1, l0_lb1, l0_lw2, l0_lb2,
               l0_gw1, l0_gb1, l0_gw2, l0_gb2)
    h = _layer(h, pos, src2d, dstg2d, dst_s, l1_lw1, l1_lb1, l1_lw2, l1_lb2,
               l1_gw1, l1_gb1, l1_gw2, l1_gb2)
    return h
